# Initial kernel scaffold; baseline (speedup 1.0000x reference)
import functools

import jax
import jax.numpy as jnp
from jax import lax
from jax.experimental import pallas as pl
from jax.experimental.pallas import tpu as pltpu
from jax.experimental.pallas import tpu_sc as plsc

V2 = 100000
K = 1000
D = 64
ROW_BLOCK = 400

BATCH, SEQ = 4096, 50
B = BATCH * SEQ
NC, NS = 2, 16
NW = NC * NS
BPW = B // NW
CHUNK = 128
NCHUNK = BPW // CHUNK


def _table_body(m_ref, l1_ref, out_ref):
    m = m_ref[...]
    mx = jnp.max(m, axis=1, keepdims=True)
    e = jnp.exp(m - mx)
    s = jnp.sum(e, axis=1, keepdims=True)
    acc = jnp.dot(e, l1_ref[...], preferred_element_type=jnp.float32)
    out_ref[...] = acc / s


def _build_table(map_weights, l1_weights):
    return pl.pallas_call(
        _table_body,
        grid=(V2 // ROW_BLOCK,),
        in_specs=[
            pl.BlockSpec((ROW_BLOCK, K), lambda i: (i, 0)),
            pl.BlockSpec((K, D), lambda i: (0, 0)),
        ],
        out_specs=pl.BlockSpec((ROW_BLOCK, D), lambda i: (i, 0)),
        out_shape=jax.ShapeDtypeStruct((V2, D), jnp.float32),
    )(map_weights, l1_weights)


def _gather_body(table_hbm, x_hbm, out_hbm, idx_v, rows_v, sem):
    wid = lax.axis_index("s") * NC + lax.axis_index("c")
    pltpu.sync_copy(x_hbm.at[wid], idx_v)
    base = wid * BPW

    def body(j, carry):
        pltpu.async_copy(table_hbm.at[idx_v.at[j]], rows_v, sem).wait()
        pltpu.sync_copy(rows_v, out_hbm.at[pl.ds(base + j * CHUNK, CHUNK)])
        return carry

    lax.fori_loop(0, NCHUNK, body, 0)


_gather = functools.partial(
    pl.kernel,
    mesh=plsc.VectorSubcoreMesh(core_axis_name="c", subcore_axis_name="s"),
    out_type=jax.ShapeDtypeStruct((B, D), jnp.float32),
    scratch_types=[
        pltpu.VMEM((NCHUNK, CHUNK), jnp.int32),
        pltpu.VMEM((CHUNK, D), jnp.float32),
        pltpu.SemaphoreType.DMA,
    ],
)(_gather_body)


def kernel(x, l1_weights, map_weights):
    table = _build_table(map_weights, l1_weights)
    idx = x.reshape(NW, NCHUNK, CHUNK).astype(jnp.int32)
    out = _gather(table, idx)
    return out.reshape(x.shape[0], x.shape[1], D)


# speedup vs baseline: 1.7016x; 1.7016x over previous
import functools

import jax
import jax.numpy as jnp
from jax import lax
from jax.experimental import pallas as pl
from jax.experimental.pallas import tpu as pltpu
from jax.experimental.pallas import tpu_sc as plsc

V2 = 100000
K = 1000
D = 64
ROW_BLOCK = 400

BATCH, SEQ = 4096, 50
B = BATCH * SEQ
NC, NS = 2, 16
NW = NC * NS
BPW = B // NW
CHUNK = 128
NCHUNK = BPW // CHUNK


def _table_body(m_ref, l1_ref, out_ref):
    m = m_ref[...]
    mx = jnp.max(m, axis=1, keepdims=True)
    e = jnp.exp(m - mx)
    s = jnp.sum(e, axis=1, keepdims=True)
    acc = jnp.dot(e, l1_ref[...], preferred_element_type=jnp.float32)
    out_ref[...] = acc / s


def _build_table(map_weights, l1_weights):
    return pl.pallas_call(
        _table_body,
        grid=(V2 // ROW_BLOCK,),
        in_specs=[
            pl.BlockSpec((ROW_BLOCK, K), lambda i: (i, 0)),
            pl.BlockSpec((K, D), lambda i: (0, 0)),
        ],
        out_specs=pl.BlockSpec((ROW_BLOCK, D), lambda i: (i, 0)),
        out_shape=jax.ShapeDtypeStruct((V2, D), jnp.float32),
    )(map_weights, l1_weights)


def _gather_body(table_hbm, x_hbm, out_hbm, idx_v, rows_v, sem):
    wid = lax.axis_index("s") * NC + lax.axis_index("c")
    pltpu.sync_copy(x_hbm.at[wid], idx_v)
    base = wid * BPW

    def body(j, carry):
        pltpu.async_copy(table_hbm.at[idx_v.at[j]], rows_v, sem).wait()
        pltpu.sync_copy(rows_v, out_hbm.at[pl.ds(base + j * CHUNK, CHUNK)])
        return carry

    lax.fori_loop(0, NCHUNK, body, 0)


_gather = functools.partial(
    pl.kernel,
    mesh=plsc.VectorSubcoreMesh(core_axis_name="c", subcore_axis_name="s"),
    out_type=jax.ShapeDtypeStruct((B, D), jnp.float32),
    scratch_types=[
        pltpu.VMEM((NCHUNK, CHUNK), jnp.int32),
        pltpu.VMEM((CHUNK, D), jnp.float32),
        pltpu.SemaphoreType.DMA,
    ],
    compiler_params=pltpu.CompilerParams(use_tc_tiling_on_sc=False),
)(_gather_body)


def kernel(x, l1_weights, map_weights):
    table = _build_table(map_weights, l1_weights)
    idx = x.reshape(NW, NCHUNK, CHUNK).astype(jnp.int32)
    out = _gather(table, idx)
    return out.reshape(x.shape[0], x.shape[1], D)


# speedup vs baseline: 1.8947x; 1.1135x over previous
import functools

import jax
import jax.numpy as jnp
from jax import lax
from jax.experimental import pallas as pl
from jax.experimental.pallas import tpu as pltpu
from jax.experimental.pallas import tpu_sc as plsc

V2 = 100000
K = 1000
D = 64
ROW_BLOCK = 1000

BATCH, SEQ = 4096, 50
B = BATCH * SEQ
NC, NS = 2, 16
NW = NC * NS
BPW = B // NW
CHUNK = 128
NCHUNK = BPW // CHUNK


def _table_body(m_ref, l1_ref, out_ref):
    m = m_ref[...]
    mx = jnp.max(m, axis=1, keepdims=True)
    e = jnp.exp(m - mx)
    s = jnp.sum(e, axis=1, keepdims=True)
    acc = jnp.dot(e, l1_ref[...], preferred_element_type=jnp.float32)
    out_ref[...] = acc / s


def _build_table(map_weights, l1_weights):
    return pl.pallas_call(
        _table_body,
        grid=(V2 // ROW_BLOCK,),
        in_specs=[
            pl.BlockSpec((ROW_BLOCK, K), lambda i: (i, 0)),
            pl.BlockSpec((K, D), lambda i: (0, 0)),
        ],
        out_specs=pl.BlockSpec((ROW_BLOCK, D), lambda i: (i, 0)),
        out_shape=jax.ShapeDtypeStruct((V2, D), jnp.float32),
    )(map_weights, l1_weights)


def _gather_body(table_hbm, x_hbm, out_hbm, idx_v, rows_v, sem):
    wid = lax.axis_index("s") * NC + lax.axis_index("c")
    pltpu.sync_copy(x_hbm.at[wid], idx_v)
    base = wid * BPW

    def body(j, carry):
        pltpu.async_copy(table_hbm.at[idx_v.at[j]], rows_v, sem).wait()
        pltpu.sync_copy(rows_v, out_hbm.at[pl.ds(base + j * CHUNK, CHUNK)])
        return carry

    lax.fori_loop(0, NCHUNK, body, 0)


_gather = functools.partial(
    pl.kernel,
    mesh=plsc.VectorSubcoreMesh(core_axis_name="c", subcore_axis_name="s"),
    out_type=jax.ShapeDtypeStruct((B, D), jnp.float32),
    scratch_types=[
        pltpu.VMEM((NCHUNK, CHUNK), jnp.int32),
        pltpu.VMEM((CHUNK, D), jnp.float32),
        pltpu.SemaphoreType.DMA,
    ],
    compiler_params=pltpu.CompilerParams(use_tc_tiling_on_sc=False),
)(_gather_body)


def kernel(x, l1_weights, map_weights):
    table = _build_table(map_weights, l1_weights)
    idx = x.reshape(NW, NCHUNK, CHUNK).astype(jnp.int32)
    out = _gather(table, idx)
    return out.reshape(x.shape[0], x.shape[1], D)


# speedup vs baseline: 1.9685x; 1.0390x over previous
import functools

import jax
import jax.numpy as jnp
from jax import lax
from jax.experimental import pallas as pl
from jax.experimental.pallas import tpu as pltpu
from jax.experimental.pallas import tpu_sc as plsc

V2 = 100000
K = 1000
D = 64
ROW_BLOCK = 2000

BATCH, SEQ = 4096, 50
B = BATCH * SEQ
NC, NS = 2, 16
NW = NC * NS
BPW = B // NW
CHUNK = 128
NCHUNK = BPW // CHUNK


def _table_body(m_ref, l1_ref, out_ref):
    m = m_ref[...]
    mx = jnp.max(m, axis=1, keepdims=True)
    e = jnp.exp(m - mx)
    s = jnp.sum(e, axis=1, keepdims=True)
    acc = jnp.dot(e, l1_ref[...], preferred_element_type=jnp.float32)
    out_ref[...] = acc / s


def _build_table(map_weights, l1_weights):
    return pl.pallas_call(
        _table_body,
        grid=(V2 // ROW_BLOCK,),
        in_specs=[
            pl.BlockSpec((ROW_BLOCK, K), lambda i: (i, 0)),
            pl.BlockSpec((K, D), lambda i: (0, 0)),
        ],
        out_specs=pl.BlockSpec((ROW_BLOCK, D), lambda i: (i, 0)),
        out_shape=jax.ShapeDtypeStruct((V2, D), jnp.float32),
    )(map_weights, l1_weights)


def _gather_body(table_hbm, x_hbm, out_hbm, idx_v, rows_v, sem):
    wid = lax.axis_index("s") * NC + lax.axis_index("c")
    pltpu.sync_copy(x_hbm.at[wid], idx_v)
    base = wid * BPW

    def body(j, carry):
        pltpu.async_copy(table_hbm.at[idx_v.at[j]], rows_v, sem).wait()
        pltpu.sync_copy(rows_v, out_hbm.at[pl.ds(base + j * CHUNK, CHUNK)])
        return carry

    lax.fori_loop(0, NCHUNK, body, 0)


_gather = functools.partial(
    pl.kernel,
    mesh=plsc.VectorSubcoreMesh(core_axis_name="c", subcore_axis_name="s"),
    out_type=jax.ShapeDtypeStruct((B, D), jnp.float32),
    scratch_types=[
        pltpu.VMEM((NCHUNK, CHUNK), jnp.int32),
        pltpu.VMEM((CHUNK, D), jnp.float32),
        pltpu.SemaphoreType.DMA,
    ],
    compiler_params=pltpu.CompilerParams(use_tc_tiling_on_sc=False),
)(_gather_body)


def kernel(x, l1_weights, map_weights):
    table = _build_table(map_weights, l1_weights)
    idx = x.reshape(NW, NCHUNK, CHUNK).astype(jnp.int32)
    out = _gather(table, idx)
    return out.reshape(x.shape[0], x.shape[1], D)


# speedup vs baseline: 1.9918x; 1.0118x over previous
import functools

import jax
import jax.numpy as jnp
from jax import lax
from jax.experimental import pallas as pl
from jax.experimental.pallas import tpu as pltpu
from jax.experimental.pallas import tpu_sc as plsc

V2 = 100000
K = 1000
D = 64
ROW_BLOCK = 4000

BATCH, SEQ = 4096, 50
B = BATCH * SEQ
NC, NS = 2, 16
NW = NC * NS
BPW = B // NW
CHUNK = 128
NCHUNK = BPW // CHUNK


def _table_body(m_ref, l1_ref, out_ref):
    m = m_ref[...]
    mx = jnp.max(m, axis=1, keepdims=True)
    e = jnp.exp(m - mx)
    s = jnp.sum(e, axis=1, keepdims=True)
    acc = jnp.dot(e, l1_ref[...], preferred_element_type=jnp.float32)
    out_ref[...] = acc / s


def _build_table(map_weights, l1_weights):
    return pl.pallas_call(
        _table_body,
        grid=(V2 // ROW_BLOCK,),
        in_specs=[
            pl.BlockSpec((ROW_BLOCK, K), lambda i: (i, 0)),
            pl.BlockSpec((K, D), lambda i: (0, 0)),
        ],
        out_specs=pl.BlockSpec((ROW_BLOCK, D), lambda i: (i, 0)),
        out_shape=jax.ShapeDtypeStruct((V2, D), jnp.float32),
    )(map_weights, l1_weights)


def _gather_body(table_hbm, x_hbm, out_hbm, idx_v, rows_v, sem):
    wid = lax.axis_index("s") * NC + lax.axis_index("c")
    pltpu.sync_copy(x_hbm.at[wid], idx_v)
    base = wid * BPW

    def body(j, carry):
        pltpu.async_copy(table_hbm.at[idx_v.at[j]], rows_v, sem).wait()
        pltpu.sync_copy(rows_v, out_hbm.at[pl.ds(base + j * CHUNK, CHUNK)])
        return carry

    lax.fori_loop(0, NCHUNK, body, 0)


_gather = functools.partial(
    pl.kernel,
    mesh=plsc.VectorSubcoreMesh(core_axis_name="c", subcore_axis_name="s"),
    out_type=jax.ShapeDtypeStruct((B, D), jnp.float32),
    scratch_types=[
        pltpu.VMEM((NCHUNK, CHUNK), jnp.int32),
        pltpu.VMEM((CHUNK, D), jnp.float32),
        pltpu.SemaphoreType.DMA,
    ],
    compiler_params=pltpu.CompilerParams(use_tc_tiling_on_sc=False),
)(_gather_body)


def kernel(x, l1_weights, map_weights):
    table = _build_table(map_weights, l1_weights)
    idx = x.reshape(NW, NCHUNK, CHUNK).astype(jnp.int32)
    out = _gather(table, idx)
    return out.reshape(x.shape[0], x.shape[1], D)
